# baseline (speedup 1.0000x reference)
import jax
import jax.numpy as jnp
from jax.experimental import pallas as pl
import jax.experimental.pallas.tpu as pltpu

EMBED_DIM = 768
NUM_HEADS = 12
NUM_EXPERTS = 8
TOP_K = 2
D_FF = 4 * EMBED_DIM
SEQ = 2048
BLK = 256
NBLK = (SEQ * TOP_K) // BLK + NUM_EXPERTS
NSLOT = NBLK * BLK
NTB = SEQ // BLK


def _moe_body(be_ref, nu_ref, t_ref, w1_ref, b1_ref, w2_ref, b2_ref,
              tok_ref, tokall_ref, gate_ref, out_ref, y_ref):
    i = pl.program_id(0)

    @pl.when(jnp.logical_and(i < nu_ref[0], i < NBLK))
    def _():
        tok_col = tok_ref[...]
        iota_row = jax.lax.broadcasted_iota(jnp.int32, (BLK, SEQ), 1)
        G = (iota_row == tok_col).astype(jnp.bfloat16)
        xg = jnp.dot(G, t_ref[...], preferred_element_type=jnp.float32)
        xb = xg.astype(jnp.bfloat16)
        z = (jnp.dot(xb, w1_ref[0], preferred_element_type=jnp.float32)
             + b1_ref[0])
        z = jax.nn.gelu(z)
        y = (jnp.dot(z.astype(jnp.bfloat16), w2_ref[0],
                     preferred_element_type=jnp.float32) + b2_ref[0])
        yg = (y * gate_ref[...]).astype(jnp.bfloat16)
        y_ref[pl.dslice(i * BLK, BLK), :] = yg

    @pl.when(jnp.logical_and(i >= nu_ref[0], i < NBLK))
    def _():
        y_ref[pl.dslice(i * BLK, BLK), :] = jnp.zeros((BLK, EMBED_DIM),
                                                      jnp.bfloat16)

    @pl.when(i >= NBLK)
    def _():
        j = i - NBLK
        tokid = j * BLK + jax.lax.broadcasted_iota(jnp.int32, (BLK, NSLOT), 0)
        A = (tokall_ref[0] == tokid).astype(jnp.bfloat16)
        out_ref[...] = jnp.dot(A, y_ref[...],
                               preferred_element_type=jnp.float32)


@jax.jit
def kernel(x, Wqkv, bqkv, Wo, bo, Wr, W1, b1, W2, b2):
    Bq, Sq, D = x.shape
    hd = D // NUM_HEADS
    qkv = x @ Wqkv + bqkv
    q, k, v = jnp.split(qkv, 3, axis=-1)

    def rs(t):
        return t.reshape(Bq, Sq, NUM_HEADS, hd).transpose(0, 2, 1, 3)

    q, k, v = rs(q), rs(k), rs(v)
    att = jnp.einsum('bhqd,bhkd->bhqk', q, k) / jnp.sqrt(hd).astype(x.dtype)
    att = jax.nn.softmax(att, axis=-1)
    o = jnp.einsum('bhqk,bhkd->bhqd', att, v).transpose(0, 2, 1, 3).reshape(Bq, Sq, D)
    h = o @ Wo + bo
    t = h.reshape(-1, D)
    logits = t @ Wr
    probs = jax.nn.softmax(logits, axis=-1)
    topv, topi = jax.lax.top_k(probs, TOP_K)
    topv = topv / jnp.sum(topv, axis=-1, keepdims=True)

    e_flat = topi.reshape(-1)
    g_flat = topv.reshape(-1)
    oh = (e_flat[:, None] == jnp.arange(NUM_EXPERTS)[None, :]).astype(jnp.float32)
    cum = jnp.cumsum(oh, axis=0)
    rank = (jnp.take_along_axis(cum, e_flat[:, None], axis=1)[:, 0]
            .astype(jnp.int32) - 1)
    cnt = cum[-1].astype(jnp.int32)
    padded_cnt = ((cnt + BLK - 1) // BLK) * BLK
    offs = jnp.concatenate(
        [jnp.zeros((1,), jnp.int32), jnp.cumsum(padded_cnt)[:-1]])
    dest = offs[e_flat] + rank
    nslots = e_flat.shape[0]
    tok = jnp.zeros((NSLOT,), jnp.int32).at[dest].set(
        jnp.arange(nslots, dtype=jnp.int32) // TOP_K)
    gate = jnp.zeros((NSLOT,), jnp.float32).at[dest].set(g_flat)
    nblk_e = padded_cnt // BLK
    blk_expert = jnp.repeat(jnp.arange(NUM_EXPERTS, dtype=jnp.int32), nblk_e,
                            total_repeat_length=NBLK)
    num_used = jnp.sum(nblk_e).reshape(1)

    grid_spec = pltpu.PrefetchScalarGridSpec(
        num_scalar_prefetch=2,
        grid=(NBLK + NTB,),
        in_specs=[
            pl.BlockSpec((Sq, D), lambda i, be, nu: (0, 0)),
            pl.BlockSpec((1, D, D_FF),
                         lambda i, be, nu: (be[jnp.minimum(i, NBLK - 1)], 0, 0)),
            pl.BlockSpec((1, 1, D_FF),
                         lambda i, be, nu: (be[jnp.minimum(i, NBLK - 1)], 0, 0)),
            pl.BlockSpec((1, D_FF, D),
                         lambda i, be, nu: (be[jnp.minimum(i, NBLK - 1)], 0, 0)),
            pl.BlockSpec((1, 1, D),
                         lambda i, be, nu: (be[jnp.minimum(i, NBLK - 1)], 0, 0)),
            pl.BlockSpec((BLK, 1),
                         lambda i, be, nu: (jnp.minimum(i, NBLK - 1), 0)),
            pl.BlockSpec((1, NSLOT), lambda i, be, nu: (0, 0)),
            pl.BlockSpec((BLK, 1),
                         lambda i, be, nu: (jnp.minimum(i, NBLK - 1), 0)),
        ],
        out_specs=pl.BlockSpec((BLK, D),
                               lambda i, be, nu: (jnp.maximum(i - NBLK, 0), 0)),
        scratch_shapes=[pltpu.VMEM((NSLOT, D), jnp.bfloat16)],
    )
    out = pl.pallas_call(
        _moe_body,
        grid_spec=grid_spec,
        out_shape=jax.ShapeDtypeStruct((Sq, D), jnp.float32),
    )(blk_expert, num_used,
      t.astype(jnp.bfloat16), W1.astype(jnp.bfloat16),
      b1.reshape(NUM_EXPERTS, 1, D_FF), W2.astype(jnp.bfloat16),
      b2.reshape(NUM_EXPERTS, 1, D),
      tok.reshape(NSLOT, 1), tok.reshape(1, NSLOT),
      gate.reshape(NSLOT, 1))

    return out.reshape(Bq, Sq, D)


# speedup vs baseline: 1.1668x; 1.1668x over previous
import jax
import jax.numpy as jnp
from jax.experimental import pallas as pl
import jax.experimental.pallas.tpu as pltpu

EMBED_DIM = 768
NUM_HEADS = 12
NUM_EXPERTS = 8
TOP_K = 2
D_FF = 4 * EMBED_DIM
SEQ = 2048
BLK = 256
NBLK = (SEQ * TOP_K) // BLK + NUM_EXPERTS
NSLOT = NBLK * BLK


def _moe_body(be_ref, nu_ref, t_ref, w1_ref, b1_ref, w2_ref, b2_ref,
              tok_ref, tokr_ref, gate_ref, out_ref, acc_ref):
    i = pl.program_id(0)

    @pl.when(i == 0)
    def _():
        acc_ref[...] = jnp.zeros_like(acc_ref)

    @pl.when(i < nu_ref[0])
    def _():
        tok_col = tok_ref[...]
        iota_row = jax.lax.broadcasted_iota(jnp.int32, (BLK, SEQ), 1)
        G = (iota_row == tok_col).astype(jnp.bfloat16)
        xg = jnp.dot(G, t_ref[...], preferred_element_type=jnp.float32)
        xb = xg.astype(jnp.bfloat16)
        z = (jnp.dot(xb, w1_ref[0].astype(jnp.bfloat16),
                     preferred_element_type=jnp.float32)
             + b1_ref[0])
        z = jax.nn.gelu(z)
        y = (jnp.dot(z.astype(jnp.bfloat16), w2_ref[0].astype(jnp.bfloat16),
                     preferred_element_type=jnp.float32) + b2_ref[0])
        yg = (y * gate_ref[...]).astype(jnp.bfloat16)
        iota_col = jax.lax.broadcasted_iota(jnp.int32, (SEQ, BLK), 0)
        GT = (iota_col == tokr_ref[0]).astype(jnp.bfloat16)
        acc_ref[...] += jnp.dot(GT, yg, preferred_element_type=jnp.float32)

    @pl.when(i == NBLK - 1)
    def _():
        out_ref[...] = acc_ref[...]


@jax.jit
def kernel(x, Wqkv, bqkv, Wo, bo, Wr, W1, b1, W2, b2):
    Bq, Sq, D = x.shape
    hd = D // NUM_HEADS
    qkv = x @ Wqkv + bqkv
    q, k, v = jnp.split(qkv, 3, axis=-1)

    def rs(t):
        return t.reshape(Bq, Sq, NUM_HEADS, hd).transpose(0, 2, 1, 3)

    q, k, v = rs(q), rs(k), rs(v)
    att = jnp.einsum('bhqd,bhkd->bhqk', q, k) / jnp.sqrt(hd).astype(x.dtype)
    att = jax.nn.softmax(att, axis=-1)
    o = jnp.einsum('bhqk,bhkd->bhqd', att, v).transpose(0, 2, 1, 3).reshape(Bq, Sq, D)
    h = o @ Wo + bo
    t = h.reshape(-1, D)
    logits = t @ Wr
    probs = jax.nn.softmax(logits, axis=-1)
    topv, topi = jax.lax.top_k(probs, TOP_K)
    topv = topv / jnp.sum(topv, axis=-1, keepdims=True)

    e_flat = topi.reshape(-1)
    g_flat = topv.reshape(-1)
    oh = (e_flat[:, None] == jnp.arange(NUM_EXPERTS)[None, :]).astype(jnp.float32)
    cum = jnp.cumsum(oh, axis=0)
    rank = (jnp.take_along_axis(cum, e_flat[:, None], axis=1)[:, 0]
            .astype(jnp.int32) - 1)
    cnt = cum[-1].astype(jnp.int32)
    padded_cnt = ((cnt + BLK - 1) // BLK) * BLK
    offs = jnp.concatenate(
        [jnp.zeros((1,), jnp.int32), jnp.cumsum(padded_cnt)[:-1]])
    dest = offs[e_flat] + rank
    nslots = e_flat.shape[0]
    tok = jnp.zeros((NSLOT,), jnp.int32).at[dest].set(
        jnp.arange(nslots, dtype=jnp.int32) // TOP_K)
    gate = jnp.zeros((NSLOT,), jnp.float32).at[dest].set(g_flat)
    nblk_e = padded_cnt // BLK
    blk_expert = jnp.repeat(jnp.arange(NUM_EXPERTS, dtype=jnp.int32), nblk_e,
                            total_repeat_length=NBLK)
    num_used = jnp.sum(nblk_e).reshape(1)

    grid_spec = pltpu.PrefetchScalarGridSpec(
        num_scalar_prefetch=2,
        grid=(NBLK,),
        in_specs=[
            pl.BlockSpec((Sq, D), lambda i, be, nu: (0, 0)),
            pl.BlockSpec((1, D, D_FF), lambda i, be, nu: (be[i], 0, 0)),
            pl.BlockSpec((1, 1, D_FF), lambda i, be, nu: (be[i], 0, 0)),
            pl.BlockSpec((1, D_FF, D), lambda i, be, nu: (be[i], 0, 0)),
            pl.BlockSpec((1, 1, D), lambda i, be, nu: (be[i], 0, 0)),
            pl.BlockSpec((BLK, 1), lambda i, be, nu: (i, 0)),
            pl.BlockSpec((1, 1, BLK), lambda i, be, nu: (i, 0, 0)),
            pl.BlockSpec((BLK, 1), lambda i, be, nu: (i, 0)),
        ],
        out_specs=pl.BlockSpec((Sq, D), lambda i, be, nu: (0, 0)),
        scratch_shapes=[pltpu.VMEM((Sq, D), jnp.float32)],
    )
    out = pl.pallas_call(
        _moe_body,
        grid_spec=grid_spec,
        out_shape=jax.ShapeDtypeStruct((Sq, D), jnp.float32),
    )(blk_expert, num_used,
      t.astype(jnp.bfloat16), W1,
      b1.reshape(NUM_EXPERTS, 1, D_FF), W2,
      b2.reshape(NUM_EXPERTS, 1, D),
      tok.reshape(NSLOT, 1), tok.reshape(NBLK, 1, BLK),
      gate.reshape(NSLOT, 1))

    return out.reshape(Bq, Sq, D)


# speedup vs baseline: 1.2257x; 1.0505x over previous
import jax
import jax.numpy as jnp
from jax.experimental import pallas as pl
import jax.experimental.pallas.tpu as pltpu

EMBED_DIM = 768
NUM_HEADS = 12
NUM_EXPERTS = 8
TOP_K = 2
D_FF = 4 * EMBED_DIM
SEQ = 2048
BLK = 256
NBLK = (SEQ * TOP_K) // BLK + NUM_EXPERTS
NSLOT = NBLK * BLK


def _moe_body(be_ref, nu_ref, t_ref, w1_ref, b1_ref, w2_ref, b2_ref,
              tok_ref, tokr_ref, gate_ref, out_ref, acc_ref):
    i = pl.program_id(0)

    @pl.when(i == 0)
    def _():
        acc_ref[...] = jnp.zeros_like(acc_ref)

    @pl.when(i < nu_ref[0])
    def _():
        tok_col = tok_ref[...]
        iota_row = jax.lax.broadcasted_iota(jnp.int32, (BLK, SEQ), 1)
        G = (iota_row == tok_col).astype(jnp.bfloat16)
        xg = jnp.dot(G, t_ref[...], preferred_element_type=jnp.float32)
        xb = xg.astype(jnp.bfloat16)
        z = (jnp.dot(xb, w1_ref[0].astype(jnp.bfloat16),
                     preferred_element_type=jnp.float32)
             + b1_ref[0])
        z = jax.nn.gelu(z)
        y = (jnp.dot(z.astype(jnp.bfloat16), w2_ref[0].astype(jnp.bfloat16),
                     preferred_element_type=jnp.float32) + b2_ref[0])
        yg = (y * gate_ref[...]).astype(jnp.bfloat16)
        iota_col = jax.lax.broadcasted_iota(jnp.int32, (SEQ, BLK), 0)
        GT = (iota_col == tokr_ref[0]).astype(jnp.bfloat16)
        acc_ref[...] += jnp.dot(GT, yg, preferred_element_type=jnp.float32)

    @pl.when(i == NBLK - 1)
    def _():
        out_ref[...] = acc_ref[...]


@jax.jit
def kernel(x, Wqkv, bqkv, Wo, bo, Wr, W1, b1, W2, b2):
    Bq, Sq, D = x.shape
    hd = D // NUM_HEADS
    qkv = x @ Wqkv + bqkv
    q, k, v = jnp.split(qkv, 3, axis=-1)

    def rs(t):
        return t.reshape(Bq, Sq, NUM_HEADS, hd).transpose(0, 2, 1, 3)

    q, k, v = rs(q), rs(k), rs(v)
    att = jnp.einsum('bhqd,bhkd->bhqk', q, k) / jnp.sqrt(hd).astype(x.dtype)
    att = jax.nn.softmax(att, axis=-1)
    o = jnp.einsum('bhqk,bhkd->bhqd', att, v).transpose(0, 2, 1, 3).reshape(Bq, Sq, D)
    h = o @ Wo + bo
    t = h.reshape(-1, D)
    logits = t @ Wr
    probs = jax.nn.softmax(logits, axis=-1)
    topv, topi = jax.lax.top_k(probs, TOP_K)
    topv = topv / jnp.sum(topv, axis=-1, keepdims=True)

    e_flat = topi.reshape(-1)
    g_flat = topv.reshape(-1)
    oh = (e_flat[:, None] == jnp.arange(NUM_EXPERTS)[None, :]).astype(jnp.float32)
    cum = jnp.cumsum(oh, axis=0)
    rank = jnp.sum(oh * cum, axis=1).astype(jnp.int32) - 1
    cnt = cum[-1].astype(jnp.int32)
    padded_cnt = ((cnt + BLK - 1) // BLK) * BLK
    offs = jnp.concatenate(
        [jnp.zeros((1,), jnp.int32), jnp.cumsum(padded_cnt)[:-1]])
    dest = (jnp.sum(oh * offs[None, :].astype(jnp.float32), axis=1)
            .astype(jnp.int32) + rank)
    nslots = e_flat.shape[0]
    packed = jnp.stack(
        [(jnp.arange(nslots, dtype=jnp.int32) // TOP_K).astype(jnp.float32),
         g_flat], axis=1)
    slotmeta = jnp.zeros((NSLOT, 2), jnp.float32).at[dest].set(packed)
    tok = slotmeta[:, 0].astype(jnp.int32)
    gate = slotmeta[:, 1]
    nblk_e = padded_cnt // BLK
    blk_expert = jnp.repeat(jnp.arange(NUM_EXPERTS, dtype=jnp.int32), nblk_e,
                            total_repeat_length=NBLK)
    num_used = jnp.sum(nblk_e).reshape(1)

    grid_spec = pltpu.PrefetchScalarGridSpec(
        num_scalar_prefetch=2,
        grid=(NBLK,),
        in_specs=[
            pl.BlockSpec((Sq, D), lambda i, be, nu: (0, 0)),
            pl.BlockSpec((1, D, D_FF), lambda i, be, nu: (be[i], 0, 0)),
            pl.BlockSpec((1, 1, D_FF), lambda i, be, nu: (be[i], 0, 0)),
            pl.BlockSpec((1, D_FF, D), lambda i, be, nu: (be[i], 0, 0)),
            pl.BlockSpec((1, 1, D), lambda i, be, nu: (be[i], 0, 0)),
            pl.BlockSpec((BLK, 1), lambda i, be, nu: (i, 0)),
            pl.BlockSpec((1, 1, BLK), lambda i, be, nu: (i, 0, 0)),
            pl.BlockSpec((BLK, 1), lambda i, be, nu: (i, 0)),
        ],
        out_specs=pl.BlockSpec((Sq, D), lambda i, be, nu: (0, 0)),
        scratch_shapes=[pltpu.VMEM((Sq, D), jnp.float32)],
    )
    out = pl.pallas_call(
        _moe_body,
        grid_spec=grid_spec,
        out_shape=jax.ShapeDtypeStruct((Sq, D), jnp.float32),
    )(blk_expert, num_used,
      t.astype(jnp.bfloat16), W1,
      b1.reshape(NUM_EXPERTS, 1, D_FF), W2,
      b2.reshape(NUM_EXPERTS, 1, D),
      tok.reshape(NSLOT, 1), tok.reshape(NBLK, 1, BLK),
      gate.reshape(NSLOT, 1))

    return out.reshape(Bq, Sq, D)
